# baseline (speedup 1.0000x reference)
import functools

import jax
import jax.numpy as jnp
from jax import lax
from jax.experimental import pallas as pl
from jax.experimental.pallas import tpu as pltpu
from jax.experimental.pallas import tpu_sc as plsc

NC = 2
NS = 16
NW = NC * NS
L = 16
CHUNK = 128

_SELU_SCALE = 1.0507009873554805
_SELU_ALPHA = 1.6732632423543772


def _selu(v):
    return _SELU_SCALE * jnp.where(v > 0, v, _SELU_ALPHA * (jnp.exp(v) - 1.0))


@functools.lru_cache(maxsize=None)
def _sc_edge_scatter(feat, n_pad, ch_tile):
    mesh = plsc.VectorSubcoreMesh(
        core_axis_name="c", subcore_axis_name="s",
        num_cores=NC, num_subcores=NS)
    rows_z = n_pad // NS // CHUNK
    half = ch_tile // 2
    out_t = jax.ShapeDtypeStruct((NC, n_pad, feat), jnp.float32)
    scratch = [
        pltpu.VMEM((half, CHUNK), jnp.int32),
        pltpu.VMEM((half, CHUNK), jnp.int32),
        pltpu.VMEM((half, CHUNK), jnp.float32),
        pltpu.VMEM((CHUNK, feat), jnp.float32),
        pltpu.VMEM((CHUNK, feat), jnp.float32),
        pltpu.VMEM_SHARED((n_pad, feat), jnp.float32),
        pltpu.SemaphoreType.DMA,
        pltpu.SemaphoreType.DMA,
        pltpu.SemaphoreType.DMA,
        pltpu.SemaphoreType.DMA,
    ]

    def body(table, src2, dst2, w2, out, idx_s, idx_d, wbuf, rows_a, rows_b,
             acc, sem_ga, sem_gb, sem_sa, sem_sb):
        c = lax.axis_index("c")
        s = lax.axis_index("s")

        def zrow(i, carry):
            for col in range(feat // L):
                rows_a[i, pl.ds(col * L, L)] = jnp.zeros((L,), jnp.float32)
            return carry
        lax.fori_loop(0, CHUNK, zrow, 0)

        def zacc(r, carry):
            pltpu.sync_copy(rows_a, acc.at[pl.ds((s * rows_z + r) * CHUNK, CHUNK)])
            return carry
        lax.fori_loop(0, rows_z, zacc, 0)
        plsc.subcore_barrier()

        def scale(rows, j):
            def scale_g(g, carry2):
                wg = wbuf[j, pl.ds(g * L, L)]
                for e16 in range(L):
                    wv = jnp.take_along_axis(
                        wg, jnp.full((L,), e16, jnp.int32), axis=0)
                    e2 = g * L + e16
                    for col in range(feat // L):
                        sl = pl.ds(col * L, L)
                        rows[e2, sl] = rows[e2, sl] * wv
                return carry2
            lax.fori_loop(0, CHUNK // L, scale_g, 0)

        t0 = (c * NS + s) * ch_tile
        for p in range(2):
            pltpu.sync_copy(src2.at[pl.ds(t0 + p * half, half)], idx_s)
            pltpu.sync_copy(dst2.at[pl.ds(t0 + p * half, half)], idx_d)
            pltpu.sync_copy(w2.at[pl.ds(t0 + p * half, half)], wbuf)

            pltpu.async_copy(table.at[idx_s.at[0]], rows_a, sem_ga)

            def pair(t, carry):
                j0 = 2 * t
                pltpu.make_async_copy(table.at[idx_s.at[j0]], rows_a, sem_ga).wait()
                scale(rows_a, j0)

                @pl.when(t > 0)
                def _drain_b():
                    pltpu.make_async_copy(
                        rows_b, acc.at[idx_d.at[j0 - 1]], sem_sb).wait()

                pltpu.async_copy(rows_a, acc.at[idx_d.at[j0]], sem_sa, add=True)
                pltpu.async_copy(table.at[idx_s.at[j0 + 1]], rows_b, sem_gb)

                pltpu.make_async_copy(table.at[idx_s.at[j0 + 1]], rows_b, sem_gb).wait()
                scale(rows_b, j0 + 1)
                pltpu.make_async_copy(rows_a, acc.at[idx_d.at[j0]], sem_sa).wait()
                pltpu.async_copy(rows_b, acc.at[idx_d.at[j0 + 1]], sem_sb, add=True)

                @pl.when(t < half // 2 - 1)
                def _more():
                    pltpu.async_copy(table.at[idx_s.at[j0 + 2]], rows_a, sem_ga)
                return carry
            lax.fori_loop(0, half // 2, pair, 0)
            pltpu.make_async_copy(rows_b, acc.at[idx_d.at[half - 1]], sem_sb).wait()
        plsc.subcore_barrier()

        def outcp(r, carry):
            off = (s * rows_z + r) * CHUNK
            pltpu.sync_copy(acc.at[pl.ds(off, CHUNK)], out.at[c, pl.ds(off, CHUNK)])
            return carry
        lax.fori_loop(0, rows_z, outcp, 0)

    return pl.kernel(body, out_type=out_t, mesh=mesh, scratch_types=scratch)


@functools.lru_cache(maxsize=None)
def _sc_degree(n_pad, ch_tile):
    mesh = plsc.VectorSubcoreMesh(
        core_axis_name="c", subcore_axis_name="s",
        num_cores=NC, num_subcores=NS)
    per_tile = n_pad // NS
    out_t = jax.ShapeDtypeStruct((NC, n_pad), jnp.float32)
    scratch = [
        pltpu.VMEM((ch_tile, CHUNK), jnp.int32),
        pltpu.VMEM((ch_tile, CHUNK), jnp.float32),
        pltpu.VMEM((per_tile,), jnp.float32),
        pltpu.VMEM_SHARED((n_pad,), jnp.float32),
    ]

    def body(dst2, w2, out, idx_d, wbuf, zbuf, acc):
        c = lax.axis_index("c")
        s = lax.axis_index("s")

        def zf(i, carry):
            zbuf[pl.ds(i * L, L)] = jnp.zeros((L,), jnp.float32)
            return carry
        lax.fori_loop(0, per_tile // L, zf, 0)
        pltpu.sync_copy(zbuf, acc.at[pl.ds(s * per_tile, per_tile)])
        plsc.subcore_barrier()

        t0 = (c * NS + s) * ch_tile
        pltpu.sync_copy(dst2.at[pl.ds(t0, ch_tile)], idx_d)
        pltpu.sync_copy(w2.at[pl.ds(t0, ch_tile)], wbuf)

        def chunk_body(j, carry):
            pltpu.sync_copy(wbuf.at[j], acc.at[idx_d.at[j]], add=True)
            return carry
        lax.fori_loop(0, ch_tile, chunk_body, 0)
        plsc.subcore_barrier()

        pltpu.sync_copy(acc.at[pl.ds(s * per_tile, per_tile)],
                        out.at[c, pl.ds(s * per_tile, per_tile)])

    return pl.kernel(body, out_type=out_t, mesh=mesh, scratch_types=scratch)


_BN = 1000


def _row_spec(f):
    return pl.BlockSpec((_BN, f), lambda i: (i, 0))


def _full_spec(a, b):
    return pl.BlockSpec((a, b), lambda i: (0, 0))


def _k1_body(x, p0, pb0, da, db, xs, r0, dinv):
    xb = x[...]
    di = lax.rsqrt(1.0 + da[...] + db[...])
    xs[...] = di * xb
    r0[...] = jnp.dot(xb, p0[...], preferred_element_type=jnp.float32) + pb0[...]
    dinv[...] = di


def _k2_body(a0, a1, xs, r0, dinv, b0, w0, w1, p1, pb1, hs1, r1):
    di = dinv[...]
    agg = a0[...] + a1[...] + xs[...]
    pre = (di * jnp.dot(agg, w0[...], preferred_element_type=jnp.float32)
           + b0[...] + r0[...])
    act = _selu(pre)
    h1 = jnp.dot(act, w1[...], preferred_element_type=jnp.float32)
    rr = jnp.dot(act, p1[...], preferred_element_type=jnp.float32) + pb1[...]
    hs1[...] = di * h1
    r1[...] = rr


def _k3_body(a10, a11, hs1, r1, dinv, b1, w2, hs2):
    di = dinv[...]
    act = _selu(di * (a10[...] + a11[...] + hs1[...]) + b1[...] + r1[...])
    val = di * jnp.dot(act, w2[...], preferred_element_type=jnp.float32)
    hs2[...] = jnp.concatenate(
        [val, jnp.zeros((val.shape[0], 128 - val.shape[1]), jnp.float32)], axis=1)


def _k4_body(a20, a21, hs2, dinv, b2, out):
    di = dinv[...]
    g = _selu(di * (a20[...] + a21[...] + hs2[...]) + b2[...])
    m = jnp.max(g, axis=-1, keepdims=True)
    ex = jnp.exp(g - m)
    out[...] = ex / jnp.sum(ex, axis=-1, keepdims=True)


def kernel(x, edge_index, edge_weight, W0, b0, W1, b1, W2, b2, P0, pb0, P1, pb1):
    n, d_in = x.shape
    e = edge_index.shape[1]
    h1w = W1.shape[1]
    kk = W2.shape[1]

    ch_min = -(-e // (NW * CHUNK))
    ch_tile = -(-ch_min // 8) * 8
    ch_total = ch_tile * NW
    e_pad = ch_total * CHUNK
    n_pad = -(-n // (NS * CHUNK)) * NS * CHUNK

    pad = e_pad - e
    pad_iota = jnp.arange(pad, dtype=jnp.int32)
    src2 = jnp.concatenate([edge_index[0], pad_iota % n]).reshape(-1, CHUNK)
    dst2 = jnp.concatenate([edge_index[1], pad_iota % n_pad]).reshape(-1, CHUNK)
    w2 = jnp.pad(edge_weight, (0, pad)).reshape(-1, CHUNK)

    b0r = b0.reshape(1, -1)
    b1r = b1.reshape(1, -1)
    b2r = b2.reshape(1, -1)
    pb0r = pb0.reshape(1, -1)
    pb1r = pb1.reshape(1, -1)

    grid = (n // _BN,)

    deg = _sc_degree(n_pad, ch_tile)(dst2, w2)
    da = deg[0, :n, None]
    db = deg[1, :n, None]

    xs, r0, dinv = pl.pallas_call(
        _k1_body,
        grid=grid,
        in_specs=[_row_spec(d_in), _full_spec(d_in, 256),
                  _full_spec(1, 256), _row_spec(1), _row_spec(1)],
        out_specs=[_row_spec(128), _row_spec(256), _row_spec(1)],
        out_shape=[jax.ShapeDtypeStruct((n, 128), jnp.float32),
                   jax.ShapeDtypeStruct((n, 256), jnp.float32),
                   jax.ShapeDtypeStruct((n, 1), jnp.float32)],
    )(x, P0, pb0r, da, db)

    scat128 = _sc_edge_scatter(128, n_pad, ch_total // NW)
    acc0 = scat128(xs, src2, dst2, w2)

    hs1, r1 = pl.pallas_call(
        _k2_body,
        grid=grid,
        in_specs=[_row_spec(128)] * 2 + [_row_spec(128), _row_spec(256),
                  _row_spec(1), _full_spec(1, 256), _full_spec(128, 256),
                  _full_spec(256, h1w), _full_spec(256, h1w), _full_spec(1, h1w)],
        out_specs=[_row_spec(h1w), _row_spec(h1w)],
        out_shape=[jax.ShapeDtypeStruct((n, h1w), jnp.float32),
                   jax.ShapeDtypeStruct((n, h1w), jnp.float32)],
    )(acc0[0, :n], acc0[1, :n], xs, r0, dinv, b0r, W0, W1, P1, pb1r)

    acc1 = scat128(hs1, src2, dst2, w2)

    hs2p = pl.pallas_call(
        _k3_body,
        grid=grid,
        in_specs=[_row_spec(h1w)] * 3 + [_row_spec(h1w), _row_spec(1),
                  _full_spec(1, h1w), _full_spec(h1w, kk)],
        out_specs=_row_spec(128),
        out_shape=jax.ShapeDtypeStruct((n, 128), jnp.float32),
    )(acc1[0, :n], acc1[1, :n], hs1, r1, dinv, b1r, W2)

    acc2 = scat128(hs2p, src2, dst2, w2)

    out = pl.pallas_call(
        _k4_body,
        grid=grid,
        in_specs=[_row_spec(kk)] * 3 + [_row_spec(1), _full_spec(1, kk)],
        out_specs=_row_spec(kk),
        out_shape=jax.ShapeDtypeStruct((n, kk), jnp.float32),
    )(acc2[0, :n, :kk], acc2[1, :n, :kk], hs2p[:, :kk], dinv, b2r)

    return out


# speedup vs baseline: 1.2463x; 1.2463x over previous
import functools

import jax
import jax.numpy as jnp
from jax import lax
from jax.experimental import pallas as pl
from jax.experimental.pallas import tpu as pltpu
from jax.experimental.pallas import tpu_sc as plsc

NC = 2
NS = 16
NW = NC * NS
L = 16
CHUNK = 128

_SELU_SCALE = 1.0507009873554805
_SELU_ALPHA = 1.6732632423543772


def _selu(v):
    return _SELU_SCALE * jnp.where(v > 0, v, _SELU_ALPHA * (jnp.exp(v) - 1.0))


@functools.lru_cache(maxsize=None)
def _sc_edge_scatter(feat, n_pad, ch_tile):
    mesh = plsc.VectorSubcoreMesh(
        core_axis_name="c", subcore_axis_name="s",
        num_cores=NC, num_subcores=NS)
    rows_z = n_pad // NS // CHUNK
    half = ch_tile // 2
    out_t = jax.ShapeDtypeStruct((NC, n_pad, feat), jnp.float32)
    scratch = [
        pltpu.VMEM((half, CHUNK), jnp.int32),
        pltpu.VMEM((half, CHUNK), jnp.int32),
        pltpu.VMEM((half, CHUNK), jnp.float32),
        pltpu.VMEM((CHUNK, feat), jnp.float32),
        pltpu.VMEM((CHUNK, feat), jnp.float32),
        pltpu.VMEM_SHARED((n_pad, feat), jnp.float32),
        pltpu.SemaphoreType.DMA,
        pltpu.SemaphoreType.DMA,
    ]

    def body(table, src2, dst2, w2, out, idx_s, idx_d, wbuf, rows_a, rows_b,
             acc, sem_ga, sem_gb):
        c = lax.axis_index("c")
        s = lax.axis_index("s")

        def zrow(i, carry):
            for col in range(feat // L):
                rows_a[i, pl.ds(col * L, L)] = jnp.zeros((L,), jnp.float32)
            return carry
        lax.fori_loop(0, CHUNK, zrow, 0)

        def zacc(r, carry):
            pltpu.sync_copy(rows_a, acc.at[pl.ds((s * rows_z + r) * CHUNK, CHUNK)])
            return carry
        lax.fori_loop(0, rows_z, zacc, 0)
        plsc.subcore_barrier()

        def scale(rows, j):
            def scale_g(g, carry2):
                wg = wbuf[j, pl.ds(g * L, L)]
                for e16 in range(L):
                    wv = jnp.take_along_axis(
                        wg, jnp.full((L,), e16, jnp.int32), axis=0)
                    e2 = g * L + e16
                    for col in range(feat // L):
                        sl = pl.ds(col * L, L)
                        rows[e2, sl] = rows[e2, sl] * wv
                return carry2
            lax.fori_loop(0, CHUNK // L, scale_g, 0)

        t0 = (c * NS + s) * ch_tile
        for p in range(2):
            pltpu.sync_copy(src2.at[pl.ds(t0 + p * half, half)], idx_s)
            pltpu.sync_copy(dst2.at[pl.ds(t0 + p * half, half)], idx_d)
            pltpu.sync_copy(w2.at[pl.ds(t0 + p * half, half)], wbuf)

            pltpu.async_copy(table.at[idx_s.at[0]], rows_a, sem_ga)

            def pair(t, carry):
                j0 = 2 * t
                pltpu.async_copy(table.at[idx_s.at[j0 + 1]], rows_b, sem_gb)
                pltpu.make_async_copy(table.at[idx_s.at[j0]], rows_a, sem_ga).wait()
                scale(rows_a, j0)
                pltpu.sync_copy(rows_a, acc.at[idx_d.at[j0]], add=True)

                @pl.when(t < half // 2 - 1)
                def _more():
                    pltpu.async_copy(table.at[idx_s.at[j0 + 2]], rows_a, sem_ga)

                pltpu.make_async_copy(table.at[idx_s.at[j0 + 1]], rows_b, sem_gb).wait()
                scale(rows_b, j0 + 1)
                pltpu.sync_copy(rows_b, acc.at[idx_d.at[j0 + 1]], add=True)
                return carry
            lax.fori_loop(0, half // 2, pair, 0)
        plsc.subcore_barrier()

        def outcp(r, carry):
            off = (s * rows_z + r) * CHUNK
            pltpu.sync_copy(acc.at[pl.ds(off, CHUNK)], out.at[c, pl.ds(off, CHUNK)])
            return carry
        lax.fori_loop(0, rows_z, outcp, 0)

    return pl.kernel(body, out_type=out_t, mesh=mesh, scratch_types=scratch)


@functools.lru_cache(maxsize=None)
def _sc_degree(n_pad, ch_tile):
    mesh = plsc.VectorSubcoreMesh(
        core_axis_name="c", subcore_axis_name="s",
        num_cores=NC, num_subcores=NS)
    per_tile = n_pad // NS
    out_t = jax.ShapeDtypeStruct((NC, n_pad), jnp.float32)
    scratch = [
        pltpu.VMEM((ch_tile, CHUNK), jnp.int32),
        pltpu.VMEM((ch_tile, CHUNK), jnp.float32),
        pltpu.VMEM((per_tile,), jnp.float32),
        pltpu.VMEM_SHARED((n_pad,), jnp.float32),
    ]

    def body(dst2, w2, out, idx_d, wbuf, zbuf, acc):
        c = lax.axis_index("c")
        s = lax.axis_index("s")

        def zf(i, carry):
            zbuf[pl.ds(i * L, L)] = jnp.zeros((L,), jnp.float32)
            return carry
        lax.fori_loop(0, per_tile // L, zf, 0)
        pltpu.sync_copy(zbuf, acc.at[pl.ds(s * per_tile, per_tile)])
        plsc.subcore_barrier()

        t0 = (c * NS + s) * ch_tile
        pltpu.sync_copy(dst2.at[pl.ds(t0, ch_tile)], idx_d)
        pltpu.sync_copy(w2.at[pl.ds(t0, ch_tile)], wbuf)

        def chunk_body(j, carry):
            pltpu.sync_copy(wbuf.at[j], acc.at[idx_d.at[j]], add=True)
            return carry
        lax.fori_loop(0, ch_tile, chunk_body, 0)
        plsc.subcore_barrier()

        pltpu.sync_copy(acc.at[pl.ds(s * per_tile, per_tile)],
                        out.at[c, pl.ds(s * per_tile, per_tile)])

    return pl.kernel(body, out_type=out_t, mesh=mesh, scratch_types=scratch)


_BN = 1000


def _row_spec(f):
    return pl.BlockSpec((_BN, f), lambda i: (i, 0))


def _full_spec(a, b):
    return pl.BlockSpec((a, b), lambda i: (0, 0))


def _k1_body(x, p0, pb0, da, db, xs, r0, dinv):
    xb = x[...]
    di = lax.rsqrt(1.0 + da[...] + db[...])
    xs[...] = di * xb
    r0[...] = jnp.dot(xb, p0[...], preferred_element_type=jnp.float32) + pb0[...]
    dinv[...] = di


def _k2_body(a0, a1, xs, r0, dinv, b0, w0, w1, p1, pb1, hs1, r1):
    di = dinv[...]
    agg = a0[...] + a1[...] + xs[...]
    pre = (di * jnp.dot(agg, w0[...], preferred_element_type=jnp.float32)
           + b0[...] + r0[...])
    act = _selu(pre)
    h1 = jnp.dot(act, w1[...], preferred_element_type=jnp.float32)
    rr = jnp.dot(act, p1[...], preferred_element_type=jnp.float32) + pb1[...]
    hs1[...] = di * h1
    r1[...] = rr


def _k3_body(a10, a11, hs1, r1, dinv, b1, w2, hs2):
    di = dinv[...]
    act = _selu(di * (a10[...] + a11[...] + hs1[...]) + b1[...] + r1[...])
    val = di * jnp.dot(act, w2[...], preferred_element_type=jnp.float32)
    hs2[...] = jnp.concatenate(
        [val, jnp.zeros((val.shape[0], 128 - val.shape[1]), jnp.float32)], axis=1)


def _k4_body(a20, a21, hs2, dinv, b2, out):
    di = dinv[...]
    g = _selu(di * (a20[...] + a21[...] + hs2[...]) + b2[...])
    m = jnp.max(g, axis=-1, keepdims=True)
    ex = jnp.exp(g - m)
    out[...] = ex / jnp.sum(ex, axis=-1, keepdims=True)


def kernel(x, edge_index, edge_weight, W0, b0, W1, b1, W2, b2, P0, pb0, P1, pb1):
    n, d_in = x.shape
    e = edge_index.shape[1]
    h1w = W1.shape[1]
    kk = W2.shape[1]

    ch_min = -(-e // (NW * CHUNK))
    ch_tile = -(-ch_min // 8) * 8
    ch_total = ch_tile * NW
    e_pad = ch_total * CHUNK
    n_pad = -(-n // (NS * CHUNK)) * NS * CHUNK

    pad = e_pad - e
    pad_iota = jnp.arange(pad, dtype=jnp.int32)
    src2 = jnp.concatenate([edge_index[0], pad_iota % n]).reshape(-1, CHUNK)
    dst2 = jnp.concatenate([edge_index[1], pad_iota % n_pad]).reshape(-1, CHUNK)
    w2 = jnp.pad(edge_weight, (0, pad)).reshape(-1, CHUNK)

    b0r = b0.reshape(1, -1)
    b1r = b1.reshape(1, -1)
    b2r = b2.reshape(1, -1)
    pb0r = pb0.reshape(1, -1)
    pb1r = pb1.reshape(1, -1)

    grid = (n // _BN,)

    deg = _sc_degree(n_pad, ch_tile)(dst2, w2)
    da = deg[0, :n, None]
    db = deg[1, :n, None]

    xs, r0, dinv = pl.pallas_call(
        _k1_body,
        grid=grid,
        in_specs=[_row_spec(d_in), _full_spec(d_in, 256),
                  _full_spec(1, 256), _row_spec(1), _row_spec(1)],
        out_specs=[_row_spec(128), _row_spec(256), _row_spec(1)],
        out_shape=[jax.ShapeDtypeStruct((n, 128), jnp.float32),
                   jax.ShapeDtypeStruct((n, 256), jnp.float32),
                   jax.ShapeDtypeStruct((n, 1), jnp.float32)],
    )(x, P0, pb0r, da, db)

    scat128 = _sc_edge_scatter(128, n_pad, ch_total // NW)
    acc0 = scat128(xs, src2, dst2, w2)

    hs1, r1 = pl.pallas_call(
        _k2_body,
        grid=grid,
        in_specs=[_row_spec(128)] * 2 + [_row_spec(128), _row_spec(256),
                  _row_spec(1), _full_spec(1, 256), _full_spec(128, 256),
                  _full_spec(256, h1w), _full_spec(256, h1w), _full_spec(1, h1w)],
        out_specs=[_row_spec(h1w), _row_spec(h1w)],
        out_shape=[jax.ShapeDtypeStruct((n, h1w), jnp.float32),
                   jax.ShapeDtypeStruct((n, h1w), jnp.float32)],
    )(acc0[0, :n], acc0[1, :n], xs, r0, dinv, b0r, W0, W1, P1, pb1r)

    acc1 = scat128(hs1, src2, dst2, w2)

    hs2p = pl.pallas_call(
        _k3_body,
        grid=grid,
        in_specs=[_row_spec(h1w)] * 3 + [_row_spec(h1w), _row_spec(1),
                  _full_spec(1, h1w), _full_spec(h1w, kk)],
        out_specs=_row_spec(128),
        out_shape=jax.ShapeDtypeStruct((n, 128), jnp.float32),
    )(acc1[0, :n], acc1[1, :n], hs1, r1, dinv, b1r, W2)

    acc2 = scat128(hs2p, src2, dst2, w2)

    out = pl.pallas_call(
        _k4_body,
        grid=grid,
        in_specs=[_row_spec(kk)] * 3 + [_row_spec(1), _full_spec(1, kk)],
        out_specs=_row_spec(kk),
        out_shape=jax.ShapeDtypeStruct((n, kk), jnp.float32),
    )(acc2[0, :n, :kk], acc2[1, :n, :kk], hs2p[:, :kk], dinv, b2r)

    return out


# speedup vs baseline: 1.2615x; 1.0122x over previous
import functools

import jax
import jax.numpy as jnp
from jax import lax
from jax.experimental import pallas as pl
from jax.experimental.pallas import tpu as pltpu
from jax.experimental.pallas import tpu_sc as plsc

NC = 2
NS = 16
NW = NC * NS
L = 16
CHUNK = 128

_SELU_SCALE = 1.0507009873554805
_SELU_ALPHA = 1.6732632423543772


def _selu(v):
    return _SELU_SCALE * jnp.where(v > 0, v, _SELU_ALPHA * (jnp.exp(v) - 1.0))


@functools.lru_cache(maxsize=None)
def _sc_edge_scatter(feat, n_pad, ch_tile):
    mesh = plsc.VectorSubcoreMesh(
        core_axis_name="c", subcore_axis_name="s",
        num_cores=NC, num_subcores=NS)
    rows_z = n_pad // NS // CHUNK
    half = ch_tile // 2
    out_t = jax.ShapeDtypeStruct((NC, n_pad, feat), jnp.float32)
    scratch = [
        pltpu.VMEM((half, CHUNK), jnp.int32),
        pltpu.VMEM((half, CHUNK), jnp.int32),
        pltpu.VMEM((half, CHUNK), jnp.float32),
        pltpu.VMEM((CHUNK, feat), jnp.float32),
        pltpu.VMEM((CHUNK, feat), jnp.float32),
        pltpu.VMEM_SHARED((n_pad, feat), jnp.float32),
        pltpu.SemaphoreType.DMA,
        pltpu.SemaphoreType.DMA,
    ]

    def body(table, src2, dst2, w2, out, idx_s, idx_d, wbuf, rows_a, rows_b,
             acc, sem_ga, sem_gb):
        c = lax.axis_index("c")
        s = lax.axis_index("s")
        t0 = (c * NS + s) * ch_tile

        pltpu.async_copy(src2.at[pl.ds(t0, half)], idx_s, sem_ga)
        pltpu.async_copy(dst2.at[pl.ds(t0, half)], idx_d, sem_ga)
        pltpu.async_copy(w2.at[pl.ds(t0, half)], wbuf, sem_ga)

        def zrow(i, carry):
            for col in range(feat // L):
                rows_a[i, pl.ds(col * L, L)] = jnp.zeros((L,), jnp.float32)
            return carry
        lax.fori_loop(0, CHUNK, zrow, 0)

        def zacc(r, carry):
            pltpu.sync_copy(rows_a, acc.at[pl.ds((s * rows_z + r) * CHUNK, CHUNK)])
            return carry
        lax.fori_loop(0, rows_z, zacc, 0)
        plsc.subcore_barrier()

        def scale(rows, j):
            def scale_g(g, carry2):
                wg = wbuf[j, pl.ds(g * L, L)]
                for e16 in range(L):
                    wv = jnp.take_along_axis(
                        wg, jnp.full((L,), e16, jnp.int32), axis=0)
                    e2 = g * L + e16
                    for col in range(feat // L):
                        sl = pl.ds(col * L, L)
                        rows[e2, sl] = rows[e2, sl] * wv
                return carry2
            lax.fori_loop(0, CHUNK // L, scale_g, 0)

        for p in range(2):
            if p == 0:
                pltpu.make_async_copy(src2.at[pl.ds(t0, half)], idx_s, sem_ga).wait()
                pltpu.make_async_copy(dst2.at[pl.ds(t0, half)], idx_d, sem_ga).wait()
                pltpu.make_async_copy(w2.at[pl.ds(t0, half)], wbuf, sem_ga).wait()
            else:
                pltpu.sync_copy(src2.at[pl.ds(t0 + p * half, half)], idx_s)
                pltpu.sync_copy(dst2.at[pl.ds(t0 + p * half, half)], idx_d)
                pltpu.sync_copy(w2.at[pl.ds(t0 + p * half, half)], wbuf)

            pltpu.async_copy(table.at[idx_s.at[0]], rows_a, sem_ga)

            def pair(t, carry):
                j0 = 2 * t
                pltpu.async_copy(table.at[idx_s.at[j0 + 1]], rows_b, sem_gb)
                pltpu.make_async_copy(table.at[idx_s.at[j0]], rows_a, sem_ga).wait()
                scale(rows_a, j0)
                pltpu.sync_copy(rows_a, acc.at[idx_d.at[j0]], add=True)

                @pl.when(t < half // 2 - 1)
                def _more():
                    pltpu.async_copy(table.at[idx_s.at[j0 + 2]], rows_a, sem_ga)

                pltpu.make_async_copy(table.at[idx_s.at[j0 + 1]], rows_b, sem_gb).wait()
                scale(rows_b, j0 + 1)
                pltpu.sync_copy(rows_b, acc.at[idx_d.at[j0 + 1]], add=True)
                return carry
            lax.fori_loop(0, half // 2, pair, 0)
        plsc.subcore_barrier()

        def outcp(r, carry):
            off = (s * rows_z + r) * CHUNK
            pltpu.sync_copy(acc.at[pl.ds(off, CHUNK)], out.at[c, pl.ds(off, CHUNK)])
            return carry
        lax.fori_loop(0, rows_z, outcp, 0)

    return pl.kernel(body, out_type=out_t, mesh=mesh, scratch_types=scratch)


@functools.lru_cache(maxsize=None)
def _sc_degree(n_pad, ch_tile):
    mesh = plsc.VectorSubcoreMesh(
        core_axis_name="c", subcore_axis_name="s",
        num_cores=NC, num_subcores=NS)
    per_tile = n_pad // NS
    out_t = jax.ShapeDtypeStruct((NC, n_pad), jnp.float32)
    scratch = [
        pltpu.VMEM((ch_tile, CHUNK), jnp.int32),
        pltpu.VMEM((ch_tile, CHUNK), jnp.float32),
        pltpu.VMEM((per_tile,), jnp.float32),
        pltpu.VMEM_SHARED((n_pad,), jnp.float32),
    ]

    def body(dst2, w2, out, idx_d, wbuf, zbuf, acc):
        c = lax.axis_index("c")
        s = lax.axis_index("s")

        def zf(i, carry):
            zbuf[pl.ds(i * L, L)] = jnp.zeros((L,), jnp.float32)
            return carry
        lax.fori_loop(0, per_tile // L, zf, 0)
        pltpu.sync_copy(zbuf, acc.at[pl.ds(s * per_tile, per_tile)])
        plsc.subcore_barrier()

        t0 = (c * NS + s) * ch_tile
        pltpu.sync_copy(dst2.at[pl.ds(t0, ch_tile)], idx_d)
        pltpu.sync_copy(w2.at[pl.ds(t0, ch_tile)], wbuf)

        def chunk_body(j, carry):
            pltpu.sync_copy(wbuf.at[j], acc.at[idx_d.at[j]], add=True)
            return carry
        lax.fori_loop(0, ch_tile, chunk_body, 0)
        plsc.subcore_barrier()

        pltpu.sync_copy(acc.at[pl.ds(s * per_tile, per_tile)],
                        out.at[c, pl.ds(s * per_tile, per_tile)])

    return pl.kernel(body, out_type=out_t, mesh=mesh, scratch_types=scratch)


_BN = 1000


def _row_spec(f):
    return pl.BlockSpec((_BN, f), lambda i: (i, 0))


def _full_spec(a, b):
    return pl.BlockSpec((a, b), lambda i: (0, 0))


def _k1_body(x, p0, pb0, da, db, xs, r0, dinv):
    xb = x[...]
    di = lax.rsqrt(1.0 + da[...] + db[...])
    xs[...] = di * xb
    r0[...] = jnp.dot(xb, p0[...], preferred_element_type=jnp.float32) + pb0[...]
    dinv[...] = di


def _k2_body(a0, a1, xs, r0, dinv, b0, w0, w1, p1, pb1, hs1, r1):
    di = dinv[...]
    agg = a0[...] + a1[...] + xs[...]
    pre = (di * jnp.dot(agg, w0[...], preferred_element_type=jnp.float32)
           + b0[...] + r0[...])
    act = _selu(pre)
    h1 = jnp.dot(act, w1[...], preferred_element_type=jnp.float32)
    rr = jnp.dot(act, p1[...], preferred_element_type=jnp.float32) + pb1[...]
    hs1[...] = di * h1
    r1[...] = rr


def _k3_body(a10, a11, hs1, r1, dinv, b1, w2, hs2):
    di = dinv[...]
    act = _selu(di * (a10[...] + a11[...] + hs1[...]) + b1[...] + r1[...])
    val = di * jnp.dot(act, w2[...], preferred_element_type=jnp.float32)
    hs2[...] = jnp.concatenate(
        [val, jnp.zeros((val.shape[0], 128 - val.shape[1]), jnp.float32)], axis=1)


def _k4_body(a20, a21, hs2, dinv, b2, out):
    di = dinv[...]
    g = _selu(di * (a20[...] + a21[...] + hs2[...]) + b2[...])
    m = jnp.max(g, axis=-1, keepdims=True)
    ex = jnp.exp(g - m)
    out[...] = ex / jnp.sum(ex, axis=-1, keepdims=True)


def kernel(x, edge_index, edge_weight, W0, b0, W1, b1, W2, b2, P0, pb0, P1, pb1):
    n, d_in = x.shape
    e = edge_index.shape[1]
    h1w = W1.shape[1]
    kk = W2.shape[1]

    ch_min = -(-e // (NW * CHUNK))
    ch_tile = -(-ch_min // 8) * 8
    ch_total = ch_tile * NW
    e_pad = ch_total * CHUNK
    n_pad = -(-n // (NS * CHUNK)) * NS * CHUNK

    pad = e_pad - e
    pad_iota = jnp.arange(pad, dtype=jnp.int32)
    src2 = jnp.concatenate([edge_index[0], pad_iota % n]).reshape(-1, CHUNK)
    dst2 = jnp.concatenate([edge_index[1], pad_iota % n_pad]).reshape(-1, CHUNK)
    w2 = jnp.pad(edge_weight, (0, pad)).reshape(-1, CHUNK)

    b0r = b0.reshape(1, -1)
    b1r = b1.reshape(1, -1)
    b2r = b2.reshape(1, -1)
    pb0r = pb0.reshape(1, -1)
    pb1r = pb1.reshape(1, -1)

    grid = (n // _BN,)

    deg = _sc_degree(n_pad, ch_tile)(dst2, w2)
    da = deg[0, :n, None]
    db = deg[1, :n, None]

    xs, r0, dinv = pl.pallas_call(
        _k1_body,
        grid=grid,
        in_specs=[_row_spec(d_in), _full_spec(d_in, 256),
                  _full_spec(1, 256), _row_spec(1), _row_spec(1)],
        out_specs=[_row_spec(128), _row_spec(256), _row_spec(1)],
        out_shape=[jax.ShapeDtypeStruct((n, 128), jnp.float32),
                   jax.ShapeDtypeStruct((n, 256), jnp.float32),
                   jax.ShapeDtypeStruct((n, 1), jnp.float32)],
    )(x, P0, pb0r, da, db)

    scat128 = _sc_edge_scatter(128, n_pad, ch_total // NW)
    acc0 = scat128(xs, src2, dst2, w2)

    hs1, r1 = pl.pallas_call(
        _k2_body,
        grid=grid,
        in_specs=[_row_spec(128)] * 2 + [_row_spec(128), _row_spec(256),
                  _row_spec(1), _full_spec(1, 256), _full_spec(128, 256),
                  _full_spec(256, h1w), _full_spec(256, h1w), _full_spec(1, h1w)],
        out_specs=[_row_spec(h1w), _row_spec(h1w)],
        out_shape=[jax.ShapeDtypeStruct((n, h1w), jnp.float32),
                   jax.ShapeDtypeStruct((n, h1w), jnp.float32)],
    )(acc0[0, :n], acc0[1, :n], xs, r0, dinv, b0r, W0, W1, P1, pb1r)

    acc1 = scat128(hs1, src2, dst2, w2)

    hs2p = pl.pallas_call(
        _k3_body,
        grid=grid,
        in_specs=[_row_spec(h1w)] * 3 + [_row_spec(h1w), _row_spec(1),
                  _full_spec(1, h1w), _full_spec(h1w, kk)],
        out_specs=_row_spec(128),
        out_shape=jax.ShapeDtypeStruct((n, 128), jnp.float32),
    )(acc1[0, :n], acc1[1, :n], hs1, r1, dinv, b1r, W2)

    acc2 = scat128(hs2p, src2, dst2, w2)

    out = pl.pallas_call(
        _k4_body,
        grid=grid,
        in_specs=[_row_spec(kk)] * 3 + [_row_spec(1), _full_spec(1, kk)],
        out_specs=_row_spec(kk),
        out_shape=jax.ShapeDtypeStruct((n, kk), jnp.float32),
    )(acc2[0, :n, :kk], acc2[1, :n, :kk], hs2p[:, :kk], dinv, b2r)

    return out


# speedup vs baseline: 1.2639x; 1.0018x over previous
import functools

import jax
import jax.numpy as jnp
from jax import lax
from jax.experimental import pallas as pl
from jax.experimental.pallas import tpu as pltpu
from jax.experimental.pallas import tpu_sc as plsc

NC = 2
NS = 16
NW = NC * NS
L = 16
CHUNK = 128

_SELU_SCALE = 1.0507009873554805
_SELU_ALPHA = 1.6732632423543772


def _selu(v):
    return _SELU_SCALE * jnp.where(v > 0, v, _SELU_ALPHA * (jnp.exp(v) - 1.0))


@functools.lru_cache(maxsize=None)
def _sc_edge_scatter(feat, n_pad, ch_tile):
    mesh = plsc.VectorSubcoreMesh(
        core_axis_name="c", subcore_axis_name="s",
        num_cores=NC, num_subcores=NS)
    rows_z = n_pad // NS // CHUNK
    half = ch_tile // 2
    out_t = jax.ShapeDtypeStruct((NC, n_pad, feat), jnp.float32)
    scratch = [
        pltpu.VMEM((half, CHUNK), jnp.int32),
        pltpu.VMEM((half, CHUNK), jnp.int32),
        pltpu.VMEM((half, CHUNK), jnp.float32),
        pltpu.VMEM((CHUNK, feat), jnp.float32),
        pltpu.VMEM((CHUNK, feat), jnp.float32),
        pltpu.VMEM_SHARED((n_pad, feat), jnp.float32),
        pltpu.SemaphoreType.DMA,
        pltpu.SemaphoreType.DMA,
    ]

    def body(table, src2, dst2, w2, out, idx_s, idx_d, wbuf, rows_a, rows_b,
             acc, sem_ga, sem_gb):
        c = lax.axis_index("c")
        s = lax.axis_index("s")
        t0 = (c * NS + s) * ch_tile

        pltpu.async_copy(src2.at[pl.ds(t0, half)], idx_s, sem_ga)
        pltpu.async_copy(dst2.at[pl.ds(t0, half)], idx_d, sem_ga)
        pltpu.async_copy(w2.at[pl.ds(t0, half)], wbuf, sem_ga)

        def zrow(i, carry):
            for col in range(feat // L):
                rows_a[i, pl.ds(col * L, L)] = jnp.zeros((L,), jnp.float32)
            return carry
        lax.fori_loop(0, CHUNK, zrow, 0)

        def zacc(r, carry):
            pltpu.sync_copy(rows_a, acc.at[pl.ds((s * rows_z + r) * CHUNK, CHUNK)])
            return carry
        lax.fori_loop(0, rows_z, zacc, 0)
        plsc.subcore_barrier()

        def scale(rows, j):
            def scale_g(g, carry2):
                wg = wbuf[j, pl.ds(g * L, L)]
                for e16 in range(L):
                    wv = jnp.take_along_axis(
                        wg, jnp.full((L,), e16, jnp.int32), axis=0)
                    e2 = g * L + e16
                    for col in range(feat // L):
                        sl = pl.ds(col * L, L)
                        rows[e2, sl] = rows[e2, sl] * wv
                return carry2
            lax.fori_loop(0, CHUNK // L, scale_g, 0)

        for p in range(2):
            if p == 0:
                pltpu.make_async_copy(src2.at[pl.ds(t0, half)], idx_s, sem_ga).wait()
                pltpu.make_async_copy(dst2.at[pl.ds(t0, half)], idx_d, sem_ga).wait()
                pltpu.make_async_copy(w2.at[pl.ds(t0, half)], wbuf, sem_ga).wait()
            else:
                pltpu.sync_copy(src2.at[pl.ds(t0 + p * half, half)], idx_s)
                pltpu.sync_copy(dst2.at[pl.ds(t0 + p * half, half)], idx_d)
                pltpu.sync_copy(w2.at[pl.ds(t0 + p * half, half)], wbuf)

            pltpu.async_copy(table.at[idx_s.at[0]], rows_a, sem_ga)

            def pair(t, carry):
                j0 = 2 * t
                pltpu.async_copy(table.at[idx_s.at[j0 + 1]], rows_b, sem_gb)
                pltpu.make_async_copy(table.at[idx_s.at[j0]], rows_a, sem_ga).wait()
                scale(rows_a, j0)
                pltpu.sync_copy(rows_a, acc.at[idx_d.at[j0]], add=True)

                @pl.when(t < half // 2 - 1)
                def _more():
                    pltpu.async_copy(table.at[idx_s.at[j0 + 2]], rows_a, sem_ga)

                pltpu.make_async_copy(table.at[idx_s.at[j0 + 1]], rows_b, sem_gb).wait()
                scale(rows_b, j0 + 1)
                pltpu.sync_copy(rows_b, acc.at[idx_d.at[j0 + 1]], add=True)
                return carry
            lax.fori_loop(0, half // 2, pair, 0)
        plsc.subcore_barrier()

        def outcp(r, carry):
            off = (s * rows_z + r) * CHUNK
            pltpu.sync_copy(acc.at[pl.ds(off, CHUNK)], out.at[c, pl.ds(off, CHUNK)])
            return carry
        lax.fori_loop(0, rows_z, outcp, 0)

    return pl.kernel(body, out_type=out_t, mesh=mesh, scratch_types=scratch)


@functools.lru_cache(maxsize=None)
def _sc_degree(n_pad, ch_tile):
    mesh = plsc.VectorSubcoreMesh(
        core_axis_name="c", subcore_axis_name="s",
        num_cores=NC, num_subcores=NS)
    per_tile = n_pad // NS
    out_t = jax.ShapeDtypeStruct((NC, n_pad), jnp.float32)
    scratch = [
        pltpu.VMEM((ch_tile, CHUNK), jnp.int32),
        pltpu.VMEM((ch_tile, CHUNK), jnp.float32),
        pltpu.VMEM((per_tile,), jnp.float32),
        pltpu.VMEM_SHARED((n_pad,), jnp.float32),
    ]

    def body(dst2, w2, out, idx_d, wbuf, zbuf, acc):
        c = lax.axis_index("c")
        s = lax.axis_index("s")

        def zf(i, carry):
            zbuf[pl.ds(i * L, L)] = jnp.zeros((L,), jnp.float32)
            return carry
        lax.fori_loop(0, per_tile // L, zf, 0)
        pltpu.sync_copy(zbuf, acc.at[pl.ds(s * per_tile, per_tile)])
        plsc.subcore_barrier()

        t0 = (c * NS + s) * ch_tile
        pltpu.sync_copy(dst2.at[pl.ds(t0, ch_tile)], idx_d)
        pltpu.sync_copy(w2.at[pl.ds(t0, ch_tile)], wbuf)

        def chunk_body(j, carry):
            pltpu.sync_copy(wbuf.at[j], acc.at[idx_d.at[j]], add=True)
            return carry
        lax.fori_loop(0, ch_tile, chunk_body, 0)
        plsc.subcore_barrier()

        pltpu.sync_copy(acc.at[pl.ds(s * per_tile, per_tile)],
                        out.at[c, pl.ds(s * per_tile, per_tile)])

    return pl.kernel(body, out_type=out_t, mesh=mesh, scratch_types=scratch)


_BN = 1000


def _row_spec(f):
    return pl.BlockSpec((_BN, f), lambda i: (i, 0))


def _full_spec(a, b):
    return pl.BlockSpec((a, b), lambda i: (0, 0))


def _k1a_body(x, p0, pb0, r0):
    r0[...] = jnp.dot(x[...], p0[...], preferred_element_type=jnp.float32) + pb0[...]


def _k1b_body(x, da, db, xs, dinv):
    di = lax.rsqrt(1.0 + da[...] + db[...])
    xs[...] = di * x[...]
    dinv[...] = di


def _k2_body(a0, a1, xs, r0, dinv, b0, w0, w1, p1, pb1, hs1, r1):
    di = dinv[...]
    agg = a0[...] + a1[...] + xs[...]
    pre = (di * jnp.dot(agg, w0[...], preferred_element_type=jnp.float32)
           + b0[...] + r0[...])
    act = _selu(pre)
    h1 = jnp.dot(act, w1[...], preferred_element_type=jnp.float32)
    rr = jnp.dot(act, p1[...], preferred_element_type=jnp.float32) + pb1[...]
    hs1[...] = di * h1
    r1[...] = rr


def _k3_body(a10, a11, hs1, r1, dinv, b1, w2, hs2):
    di = dinv[...]
    act = _selu(di * (a10[...] + a11[...] + hs1[...]) + b1[...] + r1[...])
    val = di * jnp.dot(act, w2[...], preferred_element_type=jnp.float32)
    hs2[...] = jnp.concatenate(
        [val, jnp.zeros((val.shape[0], 128 - val.shape[1]), jnp.float32)], axis=1)


def _k4_body(a20, a21, hs2, dinv, b2, out):
    di = dinv[...]
    g = _selu(di * (a20[...] + a21[...] + hs2[...]) + b2[...])
    m = jnp.max(g, axis=-1, keepdims=True)
    ex = jnp.exp(g - m)
    out[...] = ex / jnp.sum(ex, axis=-1, keepdims=True)


def kernel(x, edge_index, edge_weight, W0, b0, W1, b1, W2, b2, P0, pb0, P1, pb1):
    n, d_in = x.shape
    e = edge_index.shape[1]
    h1w = W1.shape[1]
    kk = W2.shape[1]

    ch_min = -(-e // (NW * CHUNK))
    ch_tile = -(-ch_min // 8) * 8
    ch_total = ch_tile * NW
    e_pad = ch_total * CHUNK
    n_pad = -(-n // (NS * CHUNK)) * NS * CHUNK

    pad = e_pad - e
    pad_iota = jnp.arange(pad, dtype=jnp.int32)
    src2 = jnp.concatenate([edge_index[0], pad_iota % n]).reshape(-1, CHUNK)
    dst2 = jnp.concatenate([edge_index[1], pad_iota % n_pad]).reshape(-1, CHUNK)
    w2 = jnp.pad(edge_weight, (0, pad)).reshape(-1, CHUNK)

    b0r = b0.reshape(1, -1)
    b1r = b1.reshape(1, -1)
    b2r = b2.reshape(1, -1)
    pb0r = pb0.reshape(1, -1)
    pb1r = pb1.reshape(1, -1)

    grid = (n // _BN,)

    deg = _sc_degree(n_pad, ch_tile)(dst2, w2)

    r0 = pl.pallas_call(
        _k1a_body,
        grid=grid,
        in_specs=[_row_spec(d_in), _full_spec(d_in, 256), _full_spec(1, 256)],
        out_specs=_row_spec(256),
        out_shape=jax.ShapeDtypeStruct((n, 256), jnp.float32),
    )(x, P0, pb0r)

    da = deg[0, :n, None]
    db = deg[1, :n, None]

    xs, dinv = pl.pallas_call(
        _k1b_body,
        grid=grid,
        in_specs=[_row_spec(d_in), _row_spec(1), _row_spec(1)],
        out_specs=[_row_spec(128), _row_spec(1)],
        out_shape=[jax.ShapeDtypeStruct((n, 128), jnp.float32),
                   jax.ShapeDtypeStruct((n, 1), jnp.float32)],
    )(x, da, db)

    scat128 = _sc_edge_scatter(128, n_pad, ch_total // NW)
    acc0 = scat128(xs, src2, dst2, w2)

    hs1, r1 = pl.pallas_call(
        _k2_body,
        grid=grid,
        in_specs=[_row_spec(128)] * 2 + [_row_spec(128), _row_spec(256),
                  _row_spec(1), _full_spec(1, 256), _full_spec(128, 256),
                  _full_spec(256, h1w), _full_spec(256, h1w), _full_spec(1, h1w)],
        out_specs=[_row_spec(h1w), _row_spec(h1w)],
        out_shape=[jax.ShapeDtypeStruct((n, h1w), jnp.float32),
                   jax.ShapeDtypeStruct((n, h1w), jnp.float32)],
    )(acc0[0, :n], acc0[1, :n], xs, r0, dinv, b0r, W0, W1, P1, pb1r)

    acc1 = scat128(hs1, src2, dst2, w2)

    hs2p = pl.pallas_call(
        _k3_body,
        grid=grid,
        in_specs=[_row_spec(h1w)] * 3 + [_row_spec(h1w), _row_spec(1),
                  _full_spec(1, h1w), _full_spec(h1w, kk)],
        out_specs=_row_spec(128),
        out_shape=jax.ShapeDtypeStruct((n, 128), jnp.float32),
    )(acc1[0, :n], acc1[1, :n], hs1, r1, dinv, b1r, W2)

    acc2 = scat128(hs2p, src2, dst2, w2)

    out = pl.pallas_call(
        _k4_body,
        grid=grid,
        in_specs=[_row_spec(kk)] * 3 + [_row_spec(1), _full_spec(1, kk)],
        out_specs=_row_spec(kk),
        out_shape=jax.ShapeDtypeStruct((n, kk), jnp.float32),
    )(acc2[0, :n, :kk], acc2[1, :n, :kk], hs2p[:, :kk], dinv, b2r)

    return out
